# Initial kernel scaffold; baseline (speedup 1.0000x reference)
import jax
import jax.numpy as jnp
from jax.experimental import pallas as pl


def kernel(input, embed):
    raise NotImplementedError("write your pallas kernel here")


# speedup vs baseline: 1.8551x; 1.8551x over previous
import jax
import jax.numpy as jnp
from jax.experimental import pallas as pl


def _vq_kernel(x_ref, embed_t_ref, embed_ref, q_ref, ind_ref):
    x = x_ref[0]
    embed_t = embed_t_ref[...]
    embed = embed_ref[...]

    e2 = jnp.sum(embed_t * embed_t, axis=1, keepdims=True)
    s = jnp.dot(embed_t, x, preferred_element_type=jnp.float32)
    neg = 2.0 * s - e2

    iota_k = jax.lax.broadcasted_iota(jnp.int32, neg.shape, 0)
    maxv = jnp.max(neg, axis=0, keepdims=True)
    big = jnp.int32(1 << 30)
    idx = jnp.min(jnp.where(neg == maxv, iota_k, big), axis=0)

    onehot = (iota_k == idx[None, :]).astype(jnp.float32)
    q = jnp.dot(embed, onehot, preferred_element_type=jnp.float32)

    q_ref[0] = q
    ind_ref[0, 0] = idx


def kernel(input, embed):
    b, c, h, w = input.shape
    n_codes = embed.shape[1]
    p = h * w

    x = input.reshape(b, c, p)
    embed_t = embed.T

    q, ind = pl.pallas_call(
        _vq_kernel,
        grid=(b,),
        in_specs=[
            pl.BlockSpec((1, c, p), lambda i: (i, 0, 0)),
            pl.BlockSpec((n_codes, c), lambda i: (0, 0)),
            pl.BlockSpec((c, n_codes), lambda i: (0, 0)),
        ],
        out_specs=[
            pl.BlockSpec((1, c, p), lambda i: (i, 0, 0)),
            pl.BlockSpec((1, 1, p), lambda i: (i, 0, 0)),
        ],
        out_shape=[
            jax.ShapeDtypeStruct((b, c, p), jnp.float32),
            jax.ShapeDtypeStruct((b, 1, p), jnp.int32),
        ],
    )(x, embed_t, embed)

    quantize = q.reshape(b, c, h, w)
    embed_ind = ind.reshape(b, h, w)
    return (quantize, quantize, embed_ind)


# speedup vs baseline: 2.1636x; 1.1663x over previous
import jax
import jax.numpy as jnp
from jax.experimental import pallas as pl


def _vq_kernel(x_ref, embed_t2_ref, neg_e2_ref, embed_ref, q_ref, ind_ref):
    x = x_ref[0]
    embed_t2 = embed_t2_ref[...]
    neg_e2 = neg_e2_ref[...]
    embed = embed_ref[...]

    s = jnp.dot(embed_t2, x, preferred_element_type=jnp.float32)
    neg = s + neg_e2

    idx = jnp.argmax(neg, axis=0).astype(jnp.int32)

    iota_k = jax.lax.broadcasted_iota(jnp.int32, neg.shape, 0)
    onehot = (iota_k == idx[None, :]).astype(jnp.float32)
    q = jnp.dot(embed, onehot, preferred_element_type=jnp.float32)

    q_ref[0] = q
    ind_ref[0, 0] = idx


def kernel(input, embed):
    b, c, h, w = input.shape
    n_codes = embed.shape[1]
    p = h * w

    x = input.reshape(b, c, p)
    embed_t2 = 2.0 * embed.T
    neg_e2 = -jnp.sum(embed * embed, axis=0)[:, None]

    q, ind = pl.pallas_call(
        _vq_kernel,
        grid=(b,),
        in_specs=[
            pl.BlockSpec((1, c, p), lambda i: (i, 0, 0)),
            pl.BlockSpec((n_codes, c), lambda i: (0, 0)),
            pl.BlockSpec((n_codes, 1), lambda i: (0, 0)),
            pl.BlockSpec((c, n_codes), lambda i: (0, 0)),
        ],
        out_specs=[
            pl.BlockSpec((1, c, p), lambda i: (i, 0, 0)),
            pl.BlockSpec((1, 1, p), lambda i: (i, 0, 0)),
        ],
        out_shape=[
            jax.ShapeDtypeStruct((b, c, p), jnp.float32),
            jax.ShapeDtypeStruct((b, 1, p), jnp.int32),
        ],
    )(x, embed_t2, neg_e2, embed)

    quantize = q.reshape(b, c, h, w)
    embed_ind = ind.reshape(b, h, w)
    return (quantize, quantize, embed_ind)


# speedup vs baseline: 2.2424x; 1.0364x over previous
import jax
import jax.numpy as jnp
from jax.experimental import pallas as pl


_G = 4


def _vq_kernel(x_ref, embed_t2_ref, neg_e2_ref, embed_ref, q_ref, ind_ref):
    embed_t2 = embed_t2_ref[...]
    neg_e2 = neg_e2_ref[...]
    embed = embed_ref[...]

    for g in range(_G):
        x = x_ref[g]

        s = jnp.dot(embed_t2, x, preferred_element_type=jnp.float32)
        neg = s + neg_e2

        idx = jnp.argmax(neg, axis=0).astype(jnp.int32)

        iota_k = jax.lax.broadcasted_iota(jnp.int32, neg.shape, 0)
        onehot = (iota_k == idx[None, :]).astype(jnp.float32)
        q = jnp.dot(embed, onehot, preferred_element_type=jnp.float32)

        q_ref[g] = q
        ind_ref[g, 0] = idx


def kernel(input, embed):
    b, c, h, w = input.shape
    n_codes = embed.shape[1]
    p = h * w

    x = input.reshape(b, c, p)
    embed_t2 = 2.0 * embed.T
    neg_e2 = -jnp.sum(embed * embed, axis=0)[:, None]

    q, ind = pl.pallas_call(
        _vq_kernel,
        grid=(b // _G,),
        in_specs=[
            pl.BlockSpec((_G, c, p), lambda i: (i, 0, 0)),
            pl.BlockSpec((n_codes, c), lambda i: (0, 0)),
            pl.BlockSpec((n_codes, 1), lambda i: (0, 0)),
            pl.BlockSpec((c, n_codes), lambda i: (0, 0)),
        ],
        out_specs=[
            pl.BlockSpec((_G, c, p), lambda i: (i, 0, 0)),
            pl.BlockSpec((_G, 1, p), lambda i: (i, 0, 0)),
        ],
        out_shape=[
            jax.ShapeDtypeStruct((b, c, p), jnp.float32),
            jax.ShapeDtypeStruct((b, 1, p), jnp.int32),
        ],
    )(x, embed_t2, neg_e2, embed)

    quantize = q.reshape(b, c, h, w)
    embed_ind = ind.reshape(b, h, w)
    return (quantize, quantize, embed_ind)
